# baseline (speedup 1.0000x reference)
import functools

import jax
import jax.numpy as jnp
from jax import lax
from jax.experimental import pallas as pl
from jax.experimental.pallas import tpu as pltpu
from jax.experimental.pallas import tpu_sc as plsc


def kernel(input_tensor):
    B, C, H, W = input_tensor.shape
    D = H * W
    R = B * (C // 2)
    x = input_tensor.reshape(R, 2, D)

    NC, NS = 2, 16
    NW = NC * NS
    rows_per_w = R // NW

    NB = 2
    CH = 32
    nch = rows_per_w // CH

    mesh = plsc.VectorSubcoreMesh(core_axis_name="c", subcore_axis_name="s")

    @functools.partial(
        pl.kernel,
        mesh=mesh,
        out_type=jax.ShapeDtypeStruct((R, 1, D), jnp.float32),
        scratch_types=[
            pltpu.VMEM((NB, CH, 2, D), jnp.float32),
            pltpu.SemaphoreType.DMA,
            pltpu.SemaphoreType.DMA,
            pltpu.SemaphoreType.DMA,
            pltpu.SemaphoreType.DMA,
        ],
    )
    def sel(in_hbm, out_hbm, buf, g0, g1, s0, s1):
        gsem = [g0, g1]
        ssem = [s0, s1]
        wid = lax.axis_index("s") * NC + lax.axis_index("c")
        base = wid * rows_per_w
        gath = [None] * NB
        scat = [None] * NB

        def start_scatter(k):
            b = k % NB
            return pltpu.async_copy(
                buf.at[b, :, pl.ds(0, 1), :],
                out_hbm.at[pl.ds(base + k * CH, CH)],
                ssem[b],
            )

        for k in range(nch):
            b = k % NB
            if scat[b] is not None:
                scat[b].wait()
            gath[b] = pltpu.async_copy(
                in_hbm.at[pl.ds(base + k * CH, CH)],
                buf.at[b],
                gsem[b],
            )
            if k >= 1:
                gath[(k - 1) % NB].wait()
                scat[(k - 1) % NB] = start_scatter(k - 1)
        gath[(nch - 1) % NB].wait()
        scat[(nch - 1) % NB] = start_scatter(nch - 1)
        for b in range(NB):
            if scat[b] is not None:
                scat[b].wait()

    y = sel(x)
    return y.reshape(B, C // 2, H, W)


# speedup vs baseline: 1.7338x; 1.7338x over previous
import functools

import jax
import jax.numpy as jnp
from jax.experimental import pallas as pl


def kernel(input_tensor):
    B, C, H, W = input_tensor.shape
    CO = C // 2
    x = input_tensor.reshape(B, CO, 2, H, W)

    CB = 64
    grid = (B, CO // CB)

    def body(in_ref, out_ref):
        out_ref[...] = in_ref[:, :, 0, :, :]

    y = pl.pallas_call(
        body,
        grid=grid,
        in_specs=[
            pl.BlockSpec((1, CB, 1, H, W), lambda b, j: (b, j, 0, 0, 0)),
        ],
        out_specs=pl.BlockSpec((1, CB, H, W), lambda b, j: (b, j, 0, 0)),
        out_shape=jax.ShapeDtypeStruct((B, CO, H, W), jnp.float32),
    )(x)
    return y
